# Initial kernel scaffold; baseline (speedup 1.0000x reference)
import jax
import jax.numpy as jnp
from jax.experimental import pallas as pl


def kernel(temporal_block, mask_token, revert_idx):
    raise NotImplementedError("write your pallas kernel here")


# speedup vs baseline: 3.2460x; 3.2460x over previous
import functools

import jax
import jax.numpy as jnp
from jax import lax
from jax.experimental import pallas as pl
from jax.experimental.pallas import tpu as pltpu
from jax.experimental.pallas import tpu_sc as plsc

_NC = 2
_NS = 16
_NW = _NC * _NS
_L = 16

_CHUNK = 128


def _sc_revert(table, mask_token, ridx_flat, *, n_pos, m1, r_slots, d):
    rows_out = n_pos * (1 + r_slots)
    rows_w = rows_out // _NW
    pos_w = n_pos // _NW
    n_chunks = rows_w // _CHUNK
    d_regs = d // _L
    assert 1 + r_slots == 9 and rows_w <= 9216
    assert rows_w % _CHUNK == 0 and n_pos % _NW == 0 and d % _L == 0

    mesh = plsc.VectorSubcoreMesh(core_axis_name="c", subcore_axis_name="s")

    @functools.partial(
        pl.kernel,
        out_type=jax.ShapeDtypeStruct((rows_out, d), jnp.float32),
        mesh=mesh,
        scratch_types=[
            pltpu.VMEM((pos_w * r_slots,), jnp.int32),
            pltpu.VMEM((d,), jnp.float32),
            pltpu.VMEM((_CHUNK,), jnp.int32),
            pltpu.VMEM((_CHUNK,), jnp.int32),
            pltpu.VMEM((_CHUNK, d), jnp.float32),
            pltpu.SemaphoreType.DMA,
        ],
        compiler_params=pltpu.CompilerParams(needs_layout_passes=False),
    )
    def body(table_hbm, mtok_hbm, ridx_hbm, out_hbm,
             ridx_v, mtok_v, gidx_v, flag_v, stage_v, sem):
        wid = lax.axis_index("s") * _NC + lax.axis_index("c")
        p0 = wid * pos_w
        row_base = wid * rows_w

        pltpu.sync_copy(ridx_hbm.at[pl.ds(p0 * r_slots, pos_w * r_slots)],
                        ridx_v)
        pltpu.sync_copy(mtok_hbm, mtok_v)

        lane = lax.iota(jnp.int32, _L)

        def chunk_body(g, _):
            base = g * _CHUNK
            for v in range(_CHUNK // _L):
                nl = base + v * _L + lane
                p_loc = (nl * 7282) >> 16
                j = nl - p_loc * (1 + r_slots)
                p = p0 + p_loc
                slot = j >= 1
                off = jnp.where(slot, p_loc * r_slots + (j - 1), 0)
                rv = plsc.load_gather(ridx_v, [off])
                masked = slot & (rv >= m1 - 1)
                src = jnp.where(slot & ~masked, p * m1 + 1 + rv, p * m1)
                gidx_v[pl.ds(v * _L, _L)] = src
                flag_v[pl.ds(v * _L, _L)] = masked.astype(jnp.int32)
            pltpu.async_copy(table_hbm.at[gidx_v], stage_v, sem).wait()

            def fix_row(r, _):
                flag = plsc.load_gather(flag_v, [jnp.full((_L,), r, jnp.int32)])
                cond = flag > 0
                for db in range(d_regs):
                    cur = stage_v[r, pl.ds(db * _L, _L)]
                    mt = mtok_v[pl.ds(db * _L, _L)]
                    stage_v[r, pl.ds(db * _L, _L)] = jnp.where(cond, mt, cur)
                return 0

            lax.fori_loop(0, _CHUNK, fix_row, 0)

            pltpu.sync_copy(stage_v, out_hbm.at[pl.ds(row_base + base, _CHUNK)])
            return 0

        lax.fori_loop(0, n_chunks, chunk_body, 0)

    return body(table, mask_token, ridx_flat)


def kernel(temporal_block, mask_token, revert_idx):
    B, S, M1, D = temporal_block.shape
    R = revert_idx.shape[-1]
    n_pos = B * S
    table = temporal_block.reshape(n_pos * M1, D)
    ridx_flat = revert_idx.reshape(n_pos * R).astype(jnp.int32)
    out = _sc_revert(table, mask_token, ridx_flat,
                     n_pos=n_pos, m1=M1, r_slots=R, d=D)
    return out.reshape(B, S, 1 + R, D)


# speedup vs baseline: 4.7339x; 1.4584x over previous
import functools

import jax
import jax.numpy as jnp
from jax import lax
from jax.experimental import pallas as pl
from jax.experimental.pallas import tpu as pltpu
from jax.experimental.pallas import tpu_sc as plsc

_NC = 2
_NS = 16
_NW = _NC * _NS
_L = 16

_CHUNK = 128
_NBUF = 4


def _sc_revert(table, mask_token, ridx_flat, *, n_pos, m1, r_slots, d):
    rows_out = n_pos * (1 + r_slots)
    rows_w = rows_out // _NW
    pos_w = n_pos // _NW
    n_groups = rows_w // (_CHUNK * _NBUF)
    d_regs = d // _L
    assert 1 + r_slots == 9 and rows_w <= 9216
    assert rows_w % (_CHUNK * _NBUF) == 0 and n_pos % _NW == 0 and d % _L == 0

    mesh = plsc.VectorSubcoreMesh(core_axis_name="c", subcore_axis_name="s")

    @functools.partial(
        pl.kernel,
        out_type=jax.ShapeDtypeStruct((rows_out, d), jnp.float32),
        mesh=mesh,
        scratch_types=(
            [pltpu.VMEM((pos_w * r_slots,), jnp.int32),
             pltpu.VMEM((d,), jnp.float32)]
            + [pltpu.VMEM((_CHUNK,), jnp.int32)] * _NBUF
            + [pltpu.VMEM((_CHUNK,), jnp.int32)] * _NBUF
            + [pltpu.VMEM((_CHUNK, d), jnp.float32)] * _NBUF
            + [pltpu.SemaphoreType.DMA, pltpu.SemaphoreType.DMA]
        ),
        compiler_params=pltpu.CompilerParams(needs_layout_passes=False),
    )
    def body(table_hbm, mtok_hbm, ridx_hbm, out_hbm, ridx_v, mtok_v, *rest):
        gidx = rest[:_NBUF]
        flag = rest[_NBUF:2 * _NBUF]
        stage = rest[2 * _NBUF:3 * _NBUF]
        semg, semw = rest[3 * _NBUF:]

        wid = lax.axis_index("s") * _NC + lax.axis_index("c")
        p0 = wid * pos_w
        row_base = wid * rows_w

        pltpu.sync_copy(ridx_hbm.at[pl.ds(p0 * r_slots, pos_w * r_slots)],
                        ridx_v)
        pltpu.sync_copy(mtok_hbm, mtok_v)

        lane = lax.iota(jnp.int32, _L)
        mts = [mtok_v[pl.ds(db * _L, _L)] for db in range(d_regs)]

        def group_body(gg, _):
            @pl.when(gg > 0)
            def _():
                for b in range(_NBUF):
                    pltpu.make_async_copy(
                        stage[b], out_hbm.at[pl.ds(0, _CHUNK)], semw).wait()

            for b in range(_NBUF):
                base = (gg * _NBUF + b) * _CHUNK
                for v in range(_CHUNK // _L):
                    nl = base + v * _L + lane
                    p_loc = (nl * 7282) >> 16
                    j = nl - p_loc * (1 + r_slots)
                    p = p0 + p_loc
                    slot = j >= 1
                    off = jnp.where(slot, p_loc * r_slots + (j - 1), 0)
                    rv = plsc.load_gather(ridx_v, [off])
                    masked = slot & (rv >= m1 - 1)
                    src = jnp.where(slot & ~masked, p * m1 + 1 + rv, p * m1)
                    gidx[b][pl.ds(v * _L, _L)] = src
                    flag[b][pl.ds(v * _L, _L)] = masked.astype(jnp.int32)
                pltpu.async_copy(table_hbm.at[gidx[b]], stage[b], semg)

            for b in range(_NBUF):
                pltpu.make_async_copy(
                    table_hbm.at[gidx[b]], stage[b], semg).wait()

            for b in range(_NBUF):
                def fix_row(r, _, b=b):
                    fl = plsc.load_gather(
                        flag[b], [jnp.full((_L,), r, jnp.int32)])
                    cond = fl > 0
                    for db in range(d_regs):
                        cur = stage[b][r, pl.ds(db * _L, _L)]
                        stage[b][r, pl.ds(db * _L, _L)] = (
                            jnp.where(cond, mts[db], cur))
                    return 0

                lax.fori_loop(0, _CHUNK, fix_row, 0)
                base = (gg * _NBUF + b) * _CHUNK
                pltpu.async_copy(
                    stage[b], out_hbm.at[pl.ds(row_base + base, _CHUNK)], semw)
            return 0

        lax.fori_loop(0, n_groups, group_body, 0)
        for b in range(_NBUF):
            pltpu.make_async_copy(
                stage[b], out_hbm.at[pl.ds(0, _CHUNK)], semw).wait()

    return body(table, mask_token, ridx_flat)


def kernel(temporal_block, mask_token, revert_idx):
    B, S, M1, D = temporal_block.shape
    R = revert_idx.shape[-1]
    n_pos = B * S
    table = temporal_block.reshape(n_pos * M1, D)
    ridx_flat = revert_idx.reshape(n_pos * R).astype(jnp.int32)
    out = _sc_revert(table, mask_token, ridx_flat,
                     n_pos=n_pos, m1=M1, r_slots=R, d=D)
    return out.reshape(B, S, 1 + R, D)


# speedup vs baseline: 5.1867x; 1.0957x over previous
import functools

import jax
import jax.numpy as jnp
from jax import lax
from jax.experimental import pallas as pl
from jax.experimental.pallas import tpu as pltpu
from jax.experimental.pallas import tpu_sc as plsc

_NC = 2
_NS = 16
_NW = _NC * _NS
_L = 16

_NP = 16
_NBUF = 2
_SIN = 8
_SOUT = 16


def _sc_revert(temporal_block, mask_token, ridx_flat, *, b_dim, s_dim, m1,
               r_slots, d):
    n_pos = b_dim * s_dim
    pos_w = n_pos // _NW
    rows_slab = _NP * (1 + r_slots)
    n_slabs = pos_w // _NP
    d_regs = d // _L
    mrow = _NP * _SIN
    assert 1 + r_slots == 9 and rows_slab <= 9216 and rows_slab % _L == 0
    assert pos_w % _NP == 0 and d % _L == 0 and s_dim % pos_w == 0

    mesh = plsc.VectorSubcoreMesh(core_axis_name="c", subcore_axis_name="s")

    @functools.partial(
        pl.kernel,
        out_type=jax.ShapeDtypeStruct((b_dim, s_dim, 1 + r_slots, d),
                                      jnp.float32),
        mesh=mesh,
        scratch_types=(
            [pltpu.VMEM((pos_w * r_slots,), jnp.int32),
             pltpu.VMEM((d,), jnp.float32),
             pltpu.VMEM((rows_slab,), jnp.int32)]
            + [pltpu.VMEM((_NP * _SIN + 8, d), jnp.float32)] * _NBUF
            + [pltpu.VMEM((_NP * _SOUT, d), jnp.float32)] * _NBUF
            + [pltpu.SemaphoreType.DMA, pltpu.SemaphoreType.DMA]
        ),
        compiler_params=pltpu.CompilerParams(needs_layout_passes=False,
                                             use_tc_tiling_on_sc=True),
    )
    def body(tb_hbm, mtok_hbm, ridx_hbm, out_hbm, ridx_v, mtok_v, srcb, *rest):
        sin = rest[:_NBUF]
        sout = rest[_NBUF:2 * _NBUF]
        semi, semo = rest[2 * _NBUF:]

        wid = lax.axis_index("s") * _NC + lax.axis_index("c")
        p0 = wid * pos_w
        bb = p0 // s_dim
        ss0 = p0 % s_dim

        pltpu.sync_copy(ridx_hbm.at[pl.ds(p0 * r_slots, pos_w * r_slots)],
                        ridx_v)
        pltpu.sync_copy(mtok_hbm, mtok_v)
        lane = lax.iota(jnp.int32, _L)
        for b in range(_NBUF):
            for db in range(d_regs):
                sin[b][mrow, pl.ds(db * _L, _L)] = mtok_v[pl.ds(db * _L, _L)]

        def fire_in(c, b):
            for i in range(_NP):
                pltpu.async_copy(tb_hbm.at[bb, ss0 + c * _NP + i],
                                 sin[b].at[pl.ds(i * _SIN, m1)], semi)

        def drain_in(b):
            for i in range(_NP):
                pltpu.make_async_copy(tb_hbm.at[bb, ss0],
                                      sin[b].at[pl.ds(0, m1)], semi).wait()

        def fire_out(c, b):
            for i in range(_NP):
                pltpu.async_copy(sout[b].at[pl.ds(i * _SOUT, 1 + r_slots)],
                                 out_hbm.at[bb, ss0 + c * _NP + i], semo)

        def drain_out(b):
            for i in range(_NP):
                pltpu.make_async_copy(sout[b].at[pl.ds(0, 1 + r_slots)],
                                      out_hbm.at[bb, ss0], semo).wait()

        for b in range(_NBUF):
            fire_in(b, b)

        def group_body(gg, _):
            for b in range(_NBUF):
                c = gg * _NBUF + b
                drain_in(b)

                @pl.when(gg > 0)
                def _():
                    drain_out(b)

                c0 = c * _NP
                for v in range(rows_slab // _L):
                    n = v * _L + lane
                    i = (n * 7282) >> 16
                    j = n - i * (1 + r_slots)
                    slot = j >= 1
                    off = jnp.where(slot, (c0 + i) * r_slots + (j - 1), 0)
                    rv = plsc.load_gather(ridx_v, [off])
                    masked = slot & (rv >= m1 - 1)
                    row = jnp.where(slot, 1 + rv, 0)
                    srcb[pl.ds(v * _L, _L)] = jnp.where(
                        masked, mrow, i * _SIN + row)

                def build_row(r, _, b=b):
                    srow = plsc.load_gather(
                        srcb, [jnp.full((_L,), r, jnp.int32)])
                    io = (r * 7282) >> 16
                    jo = r - io * (1 + r_slots)
                    orow = io * _SOUT + jo
                    for db in range(d_regs):
                        x = plsc.load_gather(sin[b], [srow, db * _L + lane])
                        sout[b][orow, pl.ds(db * _L, _L)] = x
                    return 0

                lax.fori_loop(0, rows_slab, build_row, 0)

                fire_out(c, b)

                @pl.when(c + _NBUF < n_slabs)
                def _():
                    fire_in(c + _NBUF, b)
            return 0

        lax.fori_loop(0, n_slabs // _NBUF, group_body, 0)
        for b in range(_NBUF):
            drain_out(b)

    return body(temporal_block, mask_token, ridx_flat)


def kernel(temporal_block, mask_token, revert_idx):
    B, S, M1, D = temporal_block.shape
    R = revert_idx.shape[-1]
    ridx_flat = revert_idx.reshape(B * S * R).astype(jnp.int32)
    return _sc_revert(temporal_block, mask_token, ridx_flat,
                      b_dim=B, s_dim=S, m1=M1, r_slots=R, d=D)


# speedup vs baseline: 6.4642x; 1.2463x over previous
import functools

import jax
import jax.numpy as jnp
from jax import lax
from jax.experimental import pallas as pl
from jax.experimental.pallas import tpu as pltpu
from jax.experimental.pallas import tpu_sc as plsc

_NC = 2
_NS = 16
_NW = _NC * _NS
_L = 16

_NP = 16
_ROWS = 144
_NBUF = 4
_SOUT = 16


def _sc_revert(table8, ridx_flat, *, n_pos, r_slots, d):
    pos_w = n_pos // _NW
    n_slabs = pos_w // _NP
    assert 1 + r_slots == 9 and _ROWS == _NP * 9 and r_slots == 8
    assert pos_w % (_NP * _NBUF) == 0 and d % _L == 0

    mesh = plsc.VectorSubcoreMesh(core_axis_name="c", subcore_axis_name="s")

    @functools.partial(
        pl.kernel,
        out_type=jax.ShapeDtypeStruct((n_pos * _SOUT, d), jnp.float32),
        mesh=mesh,
        scratch_types=(
            [pltpu.VMEM((pos_w * r_slots,), jnp.int32)]
            + [pltpu.VMEM((_ROWS,), jnp.int32)] * _NBUF
            + [pltpu.VMEM((_ROWS, d), jnp.float32)] * _NBUF
            + [pltpu.SemaphoreType.DMA, pltpu.SemaphoreType.DMA]
        ),
        compiler_params=pltpu.CompilerParams(needs_layout_passes=False),
    )
    def body(tb_hbm, ridx_hbm, out_hbm, ridx_v, *rest):
        gidx = rest[:_NBUF]
        stage = rest[_NBUF:2 * _NBUF]
        semg, semo = rest[2 * _NBUF:]

        wid = lax.axis_index("s") * _NC + lax.axis_index("c")
        p0 = wid * pos_w

        pltpu.sync_copy(ridx_hbm.at[pl.ds(p0 * r_slots, pos_w * r_slots)],
                        ridx_v)

        lane = lax.iota(jnp.int32, _L)

        def drain_out(b):
            for i in range(_NP):
                pltpu.make_async_copy(stage[b].at[pl.ds(0, 8)],
                                      out_hbm.at[pl.ds(0, 8)], semo).wait()
                pltpu.make_async_copy(stage[b].at[128],
                                      out_hbm.at[0], semo).wait()

        def group_body(gg, _):
            for b in range(_NBUF):
                c0 = (gg * _NBUF + b) * _NP
                for v in range(8):
                    n = v * _L + lane
                    i = n >> 3
                    j = n & 7
                    slot = j >= 1
                    off = jnp.where(slot, (c0 + i) * r_slots + (j - 1), 0)
                    rv = plsc.load_gather(ridx_v, [off])
                    row = jnp.where(slot, 1 + jnp.minimum(rv, 6), 0)
                    gidx[b][pl.ds(v * _L, _L)] = (p0 + c0 + i) * 8 + row
                off8 = (c0 + lane) * r_slots + (r_slots - 1)
                rv8 = plsc.load_gather(ridx_v, [off8])
                gidx[b][pl.ds(128, _L)] = (
                    (p0 + c0 + lane) * 8 + 1 + jnp.minimum(rv8, 6))
                pltpu.async_copy(tb_hbm.at[gidx[b].at[pl.ds(0, 128)]],
                                 stage[b].at[pl.ds(0, 128)], semg)
                pltpu.async_copy(tb_hbm.at[gidx[b].at[pl.ds(128, _L)]],
                                 stage[b].at[pl.ds(128, _L)], semg)

            for b in range(_NBUF):
                c0 = (gg * _NBUF + b) * _NP
                pltpu.make_async_copy(tb_hbm.at[gidx[b].at[pl.ds(0, 128)]],
                                      stage[b].at[pl.ds(0, 128)], semg).wait()
                pltpu.make_async_copy(tb_hbm.at[gidx[b].at[pl.ds(128, _L)]],
                                      stage[b].at[pl.ds(128, _L)], semg).wait()

                @pl.when(gg > 0)
                def _():
                    drain_out(b)

                for i in range(_NP):
                    obase = (p0 + c0 + i) * _SOUT
                    pltpu.async_copy(stage[b].at[pl.ds(i * 8, 8)],
                                     out_hbm.at[pl.ds(obase, 8)], semo)
                    pltpu.async_copy(stage[b].at[128 + i],
                                     out_hbm.at[obase + 8], semo)
            return 0

        lax.fori_loop(0, n_slabs // _NBUF, group_body, 0)
        for b in range(_NBUF):
            drain_out(b)

    return body(table8, ridx_flat)


def kernel(temporal_block, mask_token, revert_idx):
    B, S, M1, D = temporal_block.shape
    R = revert_idx.shape[-1]
    n_pos = B * S
    tb5 = temporal_block.reshape(n_pos, M1, D)
    maskrows = jnp.broadcast_to(mask_token[None, None, :], (n_pos, 8 - M1, D))
    table8 = jnp.concatenate([tb5, maskrows], axis=1).reshape(n_pos * 8, D)
    ridx_flat = revert_idx.reshape(n_pos * R).astype(jnp.int32)
    out16 = _sc_revert(table8, ridx_flat, n_pos=n_pos, r_slots=R, d=D)
    return out16.reshape(n_pos, _SOUT, D)[:, :1 + R, :].reshape(B, S, 1 + R, D)
